# Initial kernel scaffold; baseline (speedup 1.0000x reference)
import jax
import jax.numpy as jnp
from jax.experimental import pallas as pl


def kernel(X):
    raise NotImplementedError("write your pallas kernel here")


# speedup vs baseline: 11.1536x; 11.1536x over previous
import functools

import jax
import jax.numpy as jnp
from jax import lax
from jax.experimental import pallas as pl
from jax.experimental.pallas import tpu as pltpu

_ZERO_RATIO = 0.9


def _select_mask_body(k_rank, x_ref, o_ref):
    _TOP = jnp.int32(-2147483648)
    x = x_ref[...]
    b = lax.bitcast_convert_type(x, jnp.int32)
    s = jnp.where(b >= 0, b, (~b) ^ _TOP)
    rows = x.shape[0]
    res = jnp.zeros((rows, 1), jnp.int32)
    k = jnp.int32(k_rank)
    for bit in range(31, -1, -1):
        if bit == 31:
            cand = res | _TOP
        else:
            cand = res | jnp.int32(1 << bit)
        thresh = cand ^ _TOP
        cnt = jnp.sum((s < thresh).astype(jnp.int32), axis=1, keepdims=True)
        res = jnp.where(cnt <= k, cand, res)
    theta_bits = jnp.where(res < 0, res ^ _TOP, ~res)
    theta = lax.bitcast_convert_type(theta_bits, jnp.float32)
    o_ref[...] = jnp.where(x > theta, x, 0.0)


def kernel(X):
    batch, n = X.shape
    k_rank = int((1.0 - _ZERO_RATIO) * n)
    rows_per_block = 8
    grid = batch // rows_per_block
    return pl.pallas_call(
        functools.partial(_select_mask_body, k_rank),
        grid=(grid,),
        in_specs=[pl.BlockSpec((rows_per_block, n), lambda i: (i, 0))],
        out_specs=pl.BlockSpec((rows_per_block, n), lambda i: (i, 0)),
        out_shape=jax.ShapeDtypeStruct((batch, n), X.dtype),
    )(X)
